# baseline (speedup 1.0000x reference)
import functools

import jax
import jax.numpy as jnp
from jax import lax
from jax.experimental import pallas as pl
from jax.experimental.pallas import tpu as pltpu
from jax.experimental.pallas import tpu_sc as plsc

NTILE = 16
NSC = 2
NW = NTILE * NSC
CK = 128
WH = 8


def _sc_mesh():
    return plsc.VectorSubcoreMesh(core_axis_name="c", subcore_axis_name="s")


def _make_hist(steps, NP):
    rpt = NP // NTILE

    @functools.partial(
        pl.kernel,
        out_type=jax.ShapeDtypeStruct((NSC, NP, WH), jnp.float32),
        mesh=_sc_mesh(),
        scratch_types=[
            pltpu.VMEM((steps, CK), jnp.int32),
            pltpu.VMEM((CK, WH), jnp.float32),
            pltpu.VMEM_SHARED((NP, WH), jnp.float32),
        ],
        compiler_params=pltpu.CompilerParams(use_tc_tiling_on_sc=False),
    )
    def hist_k(dst_hbm, ones_hbm, zeros_hbm, out_hbm, dst_v, ones_v, acc):
        c = lax.axis_index("c")
        s = lax.axis_index("s")
        w = c * NTILE + s
        pltpu.sync_copy(dst_hbm.at[w], dst_v)
        pltpu.sync_copy(ones_hbm, ones_v)
        pltpu.sync_copy(zeros_hbm, acc.at[pl.ds(s * rpt, rpt)])
        plsc.subcore_barrier()

        def body(j, carry):
            pltpu.sync_copy(ones_v, acc.at[dst_v.at[j]], add=True)
            return carry

        lax.fori_loop(0, steps, body, 0)
        plsc.subcore_barrier()
        pltpu.sync_copy(acc.at[pl.ds(s * rpt, rpt)],
                        out_hbm.at[c].at[pl.ds(s * rpt, rpt)])

    return hist_k


def _make_edges(steps, NP, CH):
    rpt = NP // NTILE
    nchunks = steps // CH

    @functools.partial(
        pl.kernel,
        out_type=jax.ShapeDtypeStruct((NSC, NP, 128), jnp.bfloat16),
        mesh=_sc_mesh(),
        scratch_types=[
            pltpu.VMEM((CH, CK), jnp.int32),
            pltpu.VMEM((CH, CK), jnp.int32),
            pltpu.VMEM((CK, 64), jnp.bfloat16),
            pltpu.VMEM((CK, 64), jnp.bfloat16),
            pltpu.VMEM((CK, 64), jnp.bfloat16),
            pltpu.VMEM((CK, 64), jnp.bfloat16),
            pltpu.VMEM_SHARED((NP, 64), jnp.bfloat16),
            pltpu.SemaphoreType.DMA,
            pltpu.SemaphoreType.DMA,
            pltpu.SemaphoreType.DMA,
            pltpu.SemaphoreType.DMA,
        ],
        compiler_params=pltpu.CompilerParams(use_tc_tiling_on_sc=False),
    )
    def edges_k(src_hbm, dst_hbm, t0, t1, zeros_hbm, out_hbm,
                src_c, dst_c, r0, r1, r2, r3, acc, s0, s1, s2, s3):
        c = lax.axis_index("c")
        s = lax.axis_index("s")
        w = c * NTILE + s
        rows = (r0, r1, r2, r3)
        sems = (s0, s1, s2, s3)
        for g, tbl in enumerate((t0, t1)):
            pltpu.sync_copy(zeros_hbm, acc.at[pl.ds(s * rpt, rpt)])
            plsc.subcore_barrier()

            def chunk(ci, carry):
                pltpu.sync_copy(src_hbm.at[w].at[pl.ds(ci * CH, CH)], src_c)
                pltpu.sync_copy(dst_hbm.at[w].at[pl.ds(ci * CH, CH)], dst_c)
                for k in range(4):
                    pltpu.async_copy(tbl.at[src_c.at[k]], rows[k], sems[k])

                def quad(q, carry2):
                    for k in range(4):
                        j = 4 * q + k
                        pltpu.make_async_copy(tbl.at[src_c.at[j]], rows[k],
                                              sems[k]).wait()
                        pltpu.sync_copy(rows[k], acc.at[dst_c.at[j]],
                                        add=True)
                        pltpu.async_copy(tbl.at[src_c.at[j + 4]], rows[k],
                                         sems[k])
                    return carry2

                carry = lax.fori_loop(0, CH // 4 - 1, quad, carry)
                for k in range(4):
                    j = CH - 4 + k
                    pltpu.make_async_copy(tbl.at[src_c.at[j]], rows[k],
                                          sems[k]).wait()
                    pltpu.sync_copy(rows[k], acc.at[dst_c.at[j]], add=True)
                return carry

            lax.fori_loop(0, nchunks, chunk, 0)
            plsc.subcore_barrier()
            pltpu.sync_copy(
                acc.at[pl.ds(s * rpt, rpt)],
                out_hbm.at[c, pl.ds(s * rpt, rpt), pl.ds(64 * g, 64)])

    return edges_k


def _deg_dis(h):
    deg = (jnp.sum(h[0], axis=1, keepdims=True)
           + jnp.sum(h[1], axis=1, keepdims=True)) * (1.0 / WH) + 2.0
    return lax.rsqrt(deg)


def _scale_body(h_ref, x_ref, hc_ref, t0_ref, t1_ref):
    dis = _deg_dis(h_ref[...])
    xb = x_ref[...]
    hb = hc_ref[...]
    t0_ref[...] = (jnp.concatenate([xb[0], hb[0]], axis=1)
                   * dis).astype(jnp.bfloat16)
    t1_ref[...] = (jnp.concatenate([xb[1], hb[1]], axis=1)
                   * dis).astype(jnp.bfloat16)


def _final_body(p_ref, h_ref, x_ref, hc_ref, cc_ref,
                w_ref, b_ref, hn_ref, cn_ref):
    dis = _deg_dis(h_ref[...])
    p = p_ref[...].astype(jnp.float32)
    agg = p[0] + p[1]
    xb = x_ref[...]
    hb = hc_ref[...]
    packed = jnp.concatenate([xb[0], hb[0], xb[1], hb[1]], axis=1)
    prop = agg * dis + packed * (2.0 * dis * dis)
    wmat = w_ref[...]
    bias = b_ref[...]
    hs, cs = [], []
    for bi in range(2):
        pb = prop[:, 64 * bi:64 * (bi + 1)]
        gt = jnp.dot(pb, wmat, preferred_element_type=jnp.float32) + bias
        ii = jax.nn.sigmoid(gt[:, 0:32])
        ff = jax.nn.sigmoid(gt[:, 32:64])
        oo = jax.nn.sigmoid(gt[:, 64:96])
        gg = jnp.tanh(gt[:, 96:128])
        cn = ff * cc_ref[bi] + ii * gg
        hs.append(oo * jnp.tanh(cn))
        cs.append(cn)
    hn_ref[...] = jnp.stack(hs)
    cn_ref[...] = jnp.stack(cs)


def kernel(input_tensor, h_cur, c_cur, edge_index, W, b):
    B, N, IN = input_tensor.shape
    H = h_cur.shape[2]
    E = edge_index.shape[1]
    F = B * (IN + H)

    NP = ((N + 1 + 127) // 128) * 128
    steps = (E + NW * CK - 1) // (NW * CK)
    epad = NW * steps * CK - E

    pad_dst = N + jnp.arange(epad, dtype=jnp.int32) % (NP - N)
    src = jnp.concatenate(
        [edge_index[0], jnp.full((epad,), N, jnp.int32)]).reshape(NW, steps, CK)
    dst = jnp.concatenate(
        [edge_index[1], pad_dst]).reshape(NW, steps, CK)

    ones_h = jnp.ones((CK, WH), jnp.float32)
    zeros_h = jnp.zeros((NP // NTILE, WH), jnp.float32)
    zeros_e = jnp.zeros((NP // NTILE, 64), jnp.bfloat16)

    hist = _make_hist(steps, NP)(dst, ones_h, zeros_h)

    R = 2176
    grid = (NP // R,)
    tspec = pl.BlockSpec((R, 64), lambda i: (i, 0))
    tshape = jax.ShapeDtypeStruct((NP, 64), jnp.bfloat16)
    tables = pl.pallas_call(
        _scale_body,
        grid=grid,
        in_specs=[
            pl.BlockSpec((NSC, R, WH), lambda i: (0, i, 0)),
            pl.BlockSpec((B, R, IN), lambda i: (0, i, 0)),
            pl.BlockSpec((B, R, H), lambda i: (0, i, 0)),
        ],
        out_specs=[tspec, tspec],
        out_shape=[tshape, tshape],
    )(hist, input_tensor, h_cur)

    CH = 28
    part = _make_edges(steps, NP, CH)(src, dst, *tables, zeros_e)

    RF = 400
    fgrid = (N // RF,)
    ospec = pl.BlockSpec((B, RF, H), lambda i: (0, i, 0))
    hn, cn = pl.pallas_call(
        _final_body,
        grid=fgrid,
        in_specs=[
            pl.BlockSpec((NSC, RF, 128), lambda i: (0, i, 0)),
            pl.BlockSpec((NSC, RF, WH), lambda i: (0, i, 0)),
            pl.BlockSpec((B, RF, IN), lambda i: (0, i, 0)),
            pl.BlockSpec((B, RF, H), lambda i: (0, i, 0)),
            ospec,
            pl.BlockSpec((IN + H, 4 * H), lambda i: (0, 0)),
            pl.BlockSpec((1, 4 * H), lambda i: (0, 0)),
        ],
        out_specs=[ospec, ospec],
        out_shape=[
            jax.ShapeDtypeStruct((B, N, H), jnp.float32),
            jax.ShapeDtypeStruct((B, N, H), jnp.float32),
        ],
    )(part, hist, input_tensor, h_cur, c_cur, W, b.reshape(1, 4 * H))

    return hn, cn


# speedup vs baseline: 1.1187x; 1.1187x over previous
import functools

import jax
import jax.numpy as jnp
from jax import lax
from jax.experimental import pallas as pl
from jax.experimental.pallas import tpu as pltpu
from jax.experimental.pallas import tpu_sc as plsc

NTILE = 16
NSC = 2
NW = NTILE * NSC
CK = 128
WH = 8


def _sc_mesh():
    return plsc.VectorSubcoreMesh(core_axis_name="c", subcore_axis_name="s")


def _make_hist(steps, NP):
    rpt = NP // NTILE

    @functools.partial(
        pl.kernel,
        out_type=jax.ShapeDtypeStruct((NSC, NP, WH), jnp.float32),
        mesh=_sc_mesh(),
        scratch_types=[
            pltpu.VMEM((steps, CK), jnp.int32),
            pltpu.VMEM((CK, WH), jnp.float32),
            pltpu.VMEM_SHARED((NP, WH), jnp.float32),
        ],
        compiler_params=pltpu.CompilerParams(use_tc_tiling_on_sc=False),
    )
    def hist_k(dst_hbm, ones_hbm, zeros_hbm, out_hbm, dst_v, ones_v, acc):
        c = lax.axis_index("c")
        s = lax.axis_index("s")
        w = c * NTILE + s
        pltpu.sync_copy(dst_hbm.at[w], dst_v)
        pltpu.sync_copy(ones_hbm, ones_v)
        pltpu.sync_copy(zeros_hbm, acc.at[pl.ds(s * rpt, rpt)])
        plsc.subcore_barrier()

        def body(j, carry):
            pltpu.sync_copy(ones_v, acc.at[dst_v.at[j]], add=True)
            return carry

        lax.fori_loop(0, steps, body, 0)
        plsc.subcore_barrier()
        pltpu.sync_copy(acc.at[pl.ds(s * rpt, rpt)],
                        out_hbm.at[c].at[pl.ds(s * rpt, rpt)])

    return hist_k


def _make_edges(steps, NP, CH):
    rpt = NP // NTILE
    nchunks = steps // CH

    @functools.partial(
        pl.kernel,
        out_type=jax.ShapeDtypeStruct((NSC, NP, 128), jnp.bfloat16),
        mesh=_sc_mesh(),
        scratch_types=[
            pltpu.VMEM((CH * CK,), jnp.int32),
            pltpu.VMEM((CH, CK), jnp.int32),
            pltpu.VMEM((CK, 64), jnp.bfloat16),
            pltpu.VMEM((CK, 64), jnp.bfloat16),
            pltpu.VMEM((CK, 64), jnp.bfloat16),
            pltpu.VMEM((CK, 64), jnp.bfloat16),
            pltpu.VMEM_SHARED((NP, 64), jnp.bfloat16),
            pltpu.SemaphoreType.DMA,
            pltpu.SemaphoreType.DMA,
            pltpu.SemaphoreType.DMA,
            pltpu.SemaphoreType.DMA,
        ],
        compiler_params=pltpu.CompilerParams(use_tc_tiling_on_sc=False),
    )
    def edges_k(src_hbm, dst_hbm, t0, t1, zeros_hbm, out_hbm,
                src_c, dst_c, r0, r1, r2, r3, acc, s0, s1, s2, s3):
        c = lax.axis_index("c")
        s = lax.axis_index("s")
        w = c * NTILE + s
        rows = (r0, r1, r2, r3)
        sems = (s0, s1, s2, s3)
        for g, tbl in enumerate((t0, t1)):
            pltpu.sync_copy(zeros_hbm, acc.at[pl.ds(s * rpt, rpt)])
            plsc.subcore_barrier()

            def chunk(ci, carry):
                base = w * (steps * CK) + ci * (CH * CK)
                pltpu.sync_copy(src_hbm.at[pl.ds(base, CH * CK)], src_c)
                pltpu.sync_copy(dst_hbm.at[w].at[pl.ds(ci * CH, CH)], dst_c)

                def sidx(j):
                    return src_c.at[pl.ds(j * CK, CK)]

                for k in range(4):
                    pltpu.async_copy(tbl.at[sidx(k)], rows[k], sems[k])

                def quad(q, carry2):
                    for k in range(4):
                        j = 4 * q + k
                        pltpu.make_async_copy(tbl.at[sidx(j)], rows[k],
                                              sems[k]).wait()
                        pltpu.sync_copy(rows[k], acc.at[dst_c.at[j]],
                                        add=True)
                        pltpu.async_copy(tbl.at[sidx(j + 4)], rows[k],
                                         sems[k])
                    return carry2

                carry = lax.fori_loop(0, CH // 4 - 1, quad, carry)
                for k in range(4):
                    j = CH - 4 + k
                    pltpu.make_async_copy(tbl.at[sidx(j)], rows[k],
                                          sems[k]).wait()
                    pltpu.sync_copy(rows[k], acc.at[dst_c.at[j]], add=True)
                return carry

            lax.fori_loop(0, nchunks, chunk, 0)
            plsc.subcore_barrier()
            pltpu.sync_copy(
                acc.at[pl.ds(s * rpt, rpt)],
                out_hbm.at[c, pl.ds(s * rpt, rpt), pl.ds(64 * g, 64)])

    return edges_k


def _deg_dis(h):
    deg = (jnp.sum(h[0], axis=1, keepdims=True)
           + jnp.sum(h[1], axis=1, keepdims=True)) * (1.0 / WH) + 2.0
    return lax.rsqrt(deg)


def _scale_body(h_ref, x_ref, hc_ref, t0_ref, t1_ref):
    dis = _deg_dis(h_ref[...])
    xb = x_ref[...]
    hb = hc_ref[...]
    t0_ref[...] = (jnp.concatenate([xb[0], hb[0]], axis=1)
                   * dis).astype(jnp.bfloat16)
    t1_ref[...] = (jnp.concatenate([xb[1], hb[1]], axis=1)
                   * dis).astype(jnp.bfloat16)


def _final_body(p_ref, h_ref, t0_ref, t1_ref, cc_ref,
                w_ref, b_ref, hn_ref, cn_ref):
    dis = _deg_dis(h_ref[...])
    p = p_ref[...].astype(jnp.float32)
    agg = p[0] + p[1]
    scl = jnp.concatenate(
        [t0_ref[...], t1_ref[...]], axis=1).astype(jnp.float32)
    prop = (agg + 2.0 * scl) * dis
    wmat = w_ref[...]
    bias = b_ref[...]
    hs, cs = [], []
    for bi in range(2):
        pb = prop[:, 64 * bi:64 * (bi + 1)]
        gt = jnp.dot(pb, wmat, preferred_element_type=jnp.float32) + bias
        ii = jax.nn.sigmoid(gt[:, 0:32])
        ff = jax.nn.sigmoid(gt[:, 32:64])
        oo = jax.nn.sigmoid(gt[:, 64:96])
        gg = jnp.tanh(gt[:, 96:128])
        cn = ff * cc_ref[bi] + ii * gg
        hs.append(oo * jnp.tanh(cn))
        cs.append(cn)
    hn_ref[...] = jnp.stack(hs)
    cn_ref[...] = jnp.stack(cs)


def kernel(input_tensor, h_cur, c_cur, edge_index, W, b):
    B, N, IN = input_tensor.shape
    H = h_cur.shape[2]
    E = edge_index.shape[1]
    F = B * (IN + H)

    NP = ((N + 1 + 127) // 128) * 128
    steps = (E + NW * CK - 1) // (NW * CK)
    epad = NW * steps * CK - E

    pad_dst = N + jnp.arange(epad, dtype=jnp.int32) % (NP - N)
    src = jnp.concatenate(
        [edge_index[0], jnp.full((epad,), N, jnp.int32)])
    dst = jnp.concatenate(
        [edge_index[1], pad_dst]).reshape(NW, steps, CK)

    ones_h = jnp.ones((CK, WH), jnp.float32)
    zeros_h = jnp.zeros((NP // NTILE, WH), jnp.float32)
    zeros_e = jnp.zeros((NP // NTILE, 64), jnp.bfloat16)

    hist = _make_hist(steps, NP)(dst, ones_h, zeros_h)

    R = 2176
    grid = (NP // R,)
    tspec = pl.BlockSpec((R, 64), lambda i: (i, 0))
    tshape = jax.ShapeDtypeStruct((NP, 64), jnp.bfloat16)
    tables = pl.pallas_call(
        _scale_body,
        grid=grid,
        in_specs=[
            pl.BlockSpec((NSC, R, WH), lambda i: (0, i, 0)),
            pl.BlockSpec((B, R, IN), lambda i: (0, i, 0)),
            pl.BlockSpec((B, R, H), lambda i: (0, i, 0)),
        ],
        out_specs=[tspec, tspec],
        out_shape=[tshape, tshape],
    )(hist, input_tensor, h_cur)

    CH = 28
    part = _make_edges(steps, NP, CH)(src, dst, *tables, zeros_e)

    ospec = pl.BlockSpec((B, R, H), lambda i: (0, i, 0))
    hn, cn = pl.pallas_call(
        _final_body,
        grid=grid,
        in_specs=[
            pl.BlockSpec((NSC, R, 128), lambda i: (0, i, 0)),
            pl.BlockSpec((NSC, R, WH), lambda i: (0, i, 0)),
            tspec, tspec,
            ospec,
            pl.BlockSpec((IN + H, 4 * H), lambda i: (0, 0)),
            pl.BlockSpec((1, 4 * H), lambda i: (0, 0)),
        ],
        out_specs=[ospec, ospec],
        out_shape=[
            jax.ShapeDtypeStruct((B, N, H), jnp.float32),
            jax.ShapeDtypeStruct((B, N, H), jnp.float32),
        ],
    )(part, hist, *tables, c_cur, W, b.reshape(1, 4 * H))

    return hn, cn
